# baseline (speedup 1.0000x reference)
import functools

import jax
import jax.numpy as jnp
from jax import lax
from jax.experimental import pallas as pl
from jax.experimental.pallas import tpu as pltpu
from jax.experimental.pallas import tpu_sc as plsc

B_ROWS = 4096
SEQ = 200
D_MODEL = 64
SCALE = float(D_MODEL) ** 0.5
LANES = 16

NC, NS = 2, 16
NW = NC * NS
ROWS_W = B_ROWS // NW
NBUF = 4
NOBUF = 2
ROUNDS = ROWS_W // NBUF
G0, G1 = 128, SEQ - 128


def _tec_body(x_hbm, table_hbm, out_hbm, *sc):
    idx_v = sc[0]
    gbuf = sc[1:1 + NBUF]
    obuf = sc[1 + NBUF:1 + NBUF + NOBUF]
    gsem = sc[1 + NBUF + NOBUF:1 + 2 * NBUF + NOBUF]
    osem = sc[1 + 2 * NBUF + NOBUF:]

    wid = lax.axis_index("c") * NS + lax.axis_index("s")
    row0 = wid * ROWS_W

    pltpu.sync_copy(x_hbm.at[pl.ds(row0, ROWS_W), :], idx_v)

    def start_gather(b, g):
        pltpu.async_copy(table_hbm.at[idx_v.at[g, pl.ds(0, G0)]],
                         gbuf[b].at[pl.ds(0, G0), :], gsem[b])
        pltpu.async_copy(table_hbm.at[idx_v.at[g, pl.ds(G0, G1)]],
                         gbuf[b].at[pl.ds(G0, G1), :], gsem[b])

    def wait_gather(b):
        pltpu.make_async_copy(table_hbm.at[idx_v.at[0, pl.ds(0, G0)]],
                              gbuf[b].at[pl.ds(0, G0), :], gsem[b]).wait()
        pltpu.make_async_copy(table_hbm.at[idx_v.at[0, pl.ds(G0, G1)]],
                              gbuf[b].at[pl.ds(G0, G1), :], gsem[b]).wait()

    def start_out(ob, g):
        pltpu.async_copy(obuf[ob], out_hbm.at[row0 + g], osem[ob])

    def wait_out(ob):
        pltpu.make_async_copy(obuf[ob], out_hbm.at[row0], osem[ob]).wait()

    def scale(b, ob):
        gb, o = gbuf[b], obuf[ob]

        def body_fn(i, carry):
            r0 = i * 4
            for u in range(4):
                for j in range(D_MODEL // LANES):
                    s = pl.ds(j * LANES, LANES)
                    o[r0 + u, s] = gb[r0 + u, s] * SCALE
            return carry

        lax.fori_loop(0, SEQ // 4, body_fn, 0)

    for b in range(NBUF):
        start_gather(b, b)

    for b in range(NBUF):
        wait_gather(b)
        ob = b % NOBUF
        if b >= NOBUF:
            wait_out(ob)
        scale(b, ob)
        start_gather(b, b + NBUF)
        start_out(ob, b)

    def round_body(ro, carry):
        for b in range(NBUF):
            g = ro * NBUF + b
            wait_gather(b)
            ob = b % NOBUF
            wait_out(ob)
            scale(b, ob)
            start_gather(b, g + NBUF)
            start_out(ob, g)
        return carry

    lax.fori_loop(1, ROUNDS - 1, round_body, 0)

    for b in range(NBUF):
        g = (ROUNDS - 1) * NBUF + b
        wait_gather(b)
        ob = b % NOBUF
        wait_out(ob)
        scale(b, ob)
        start_out(ob, g)

    for ob in range(NOBUF):
        wait_out(ob)


_emb = functools.partial(
    pl.kernel,
    out_type=jax.ShapeDtypeStruct((B_ROWS, SEQ, D_MODEL), jnp.float32),
    mesh=plsc.VectorSubcoreMesh(core_axis_name="c", subcore_axis_name="s"),
    scratch_types=(
        [pltpu.VMEM((ROWS_W, SEQ), jnp.int32)]
        + [pltpu.VMEM((SEQ, D_MODEL), jnp.float32) for _ in range(NBUF + NOBUF)]
        + [pltpu.SemaphoreType.DMA for _ in range(NBUF + NOBUF)]
    ),
    compiler_params=pltpu.CompilerParams(use_tc_tiling_on_sc=False),
)(_tec_body)


def kernel(x, table):
    return _emb(x.astype(jnp.int32), table)
